# baseline (speedup 1.0000x reference)
import functools

import jax
import jax.numpy as jnp
from jax import lax
from jax.experimental import pallas as pl
from jax.experimental.pallas import tpu as pltpu
from jax.experimental.pallas import tpu_sc as plsc

_NC = 2
_NS = 16
_NW = _NC * _NS
_CH = 32
_L = 16


def _embed_sc(table, idx, pos_enc, n_rows, seq, d_model):
    mesh = plsc.VectorSubcoreMesh(core_axis_name="c", subcore_axis_name="s")
    per_w = n_rows // _NW
    n_chunks = per_w // _CH

    @functools.partial(
        pl.kernel,
        out_type=jax.ShapeDtypeStruct((n_rows, d_model), jnp.float32),
        mesh=mesh,
        scratch_types=[
            pltpu.VMEM((per_w,), jnp.int32),
            pltpu.VMEM((_CH, d_model), jnp.float32),
            pltpu.VMEM((_CH, d_model), jnp.float32),
        ],
    )
    def k(table_hbm, idx_hbm, pos_hbm, out_hbm, idx_v, rows_v, pos_v):
        wid = lax.axis_index("s") * _NC + lax.axis_index("c")
        base = wid * per_w
        pos_base = base % seq
        pltpu.sync_copy(idx_hbm.at[pl.ds(base, per_w)], idx_v)

        @pl.loop(0, n_chunks)
        def _chunk(ci):
            row0 = ci * _CH
            pltpu.sync_copy(table_hbm.at[idx_v.at[pl.ds(row0, _CH)]], rows_v)
            pltpu.sync_copy(pos_hbm.at[pl.ds(pos_base + row0, _CH)], pos_v)

            @pl.loop(0, _CH)
            def _row(r):
                @pl.loop(0, d_model, step=_L)
                def _col(c):
                    v = pos_v[r, pl.ds(c, _L)]
                    plsc.addupdate(rows_v.at[r, pl.ds(c, _L)], v)

            pltpu.sync_copy(rows_v, out_hbm.at[pl.ds(base + row0, _CH)])

    return k(table, idx, pos_enc)


def kernel(x, table, pos_enc):
    b, s = x.shape
    n = b * s
    d = table.shape[1]
    idx = x.reshape(n)
    out = _embed_sc(table, idx, pos_enc, n, s, d)
    return out.reshape(b, s, d)


# speedup vs baseline: 1.4454x; 1.4454x over previous
import functools

import jax
import jax.numpy as jnp
from jax import lax
from jax.experimental import pallas as pl
from jax.experimental.pallas import tpu as pltpu
from jax.experimental.pallas import tpu_sc as plsc

_NC = 2
_NS = 16
_NW = _NC * _NS
_CH = 16
_L = 16
_NBUF = 4


def _embed_sc(table, idx, pos_enc, batch, seq, d_model):
    mesh = plsc.VectorSubcoreMesh(core_axis_name="c", subcore_axis_name="s")
    n_rows = batch * seq
    per_w = seq // _NW
    n_chunks = per_w // _CH
    n_units = n_chunks * batch

    @functools.partial(
        pl.kernel,
        out_type=jax.ShapeDtypeStruct((n_rows, d_model), jnp.float32),
        mesh=mesh,
        scratch_types=(
            [pltpu.VMEM((batch, per_w), jnp.int32)]
            + [pltpu.VMEM((_CH, d_model), jnp.float32) for _ in range(_NBUF)]
            + [pltpu.VMEM((_CH, d_model), jnp.float32) for _ in range(2)]
            + [pltpu.SemaphoreType.DMA for _ in range(2 * _NBUF + 2)]
        ),
    )
    def k(table_hbm, idx_hbm, pos_hbm, out_hbm, idx_v,
          r0, r1, r2, r3, p0, p1,
          g0, g1, g2, g3, s0, s1, s2, s3, q0, q1):
        rows = (r0, r1, r2, r3)
        pos = (p0, p1)
        gsem = (g0, g1, g2, g3)
        ssem = (s0, s1, s2, s3)
        psem = (q0, q1)

        wid = lax.axis_index("s") * _NC + lax.axis_index("c")
        pbase = wid * per_w

        def fire_gather(u, bslot):
            ck = u // batch
            ib = bslot
            idx_ref = idx_v.at[ib, pl.ds(ck * _CH, _CH)]
            pltpu.make_async_copy(
                table_hbm.at[idx_ref], rows[bslot], gsem[bslot]).start()

        def wait_gather(bslot):
            pltpu.make_async_copy(
                table_hbm.at[idx_v.at[0, pl.ds(0, _CH)]],
                rows[bslot], gsem[bslot]).wait()

        def fire_pos(ck, pslot):
            pltpu.make_async_copy(
                pos_hbm.at[pl.ds(pbase + ck * _CH, _CH)],
                pos[pslot], psem[pslot]).start()

        def wait_pos(pslot):
            pltpu.make_async_copy(
                pos_hbm.at[pl.ds(pbase, _CH)], pos[pslot],
                psem[pslot]).wait()

        def fire_store(u, bslot):
            ck = u // batch
            b = bslot
            row0 = b * seq + pbase + ck * _CH
            pltpu.make_async_copy(
                rows[bslot], out_hbm.at[pl.ds(row0, _CH)],
                ssem[bslot]).start()

        def wait_store(bslot):
            pltpu.make_async_copy(
                rows[bslot], out_hbm.at[pl.ds(0, _CH)],
                ssem[bslot]).wait()

        for b in range(batch):
            pltpu.sync_copy(
                idx_hbm.at[pl.ds(b * seq + pbase, per_w)], idx_v.at[b])

        fire_pos(0, 0)
        fire_pos(1, 1)
        fire_gather(0, 0)
        fire_gather(1, 1)

        @pl.loop(0, n_chunks // 2)
        def _pair(pair):
            for j in range(2 * batch):
                ck = pair * 2 + (j // batch)
                bslot = j % batch
                pslot = j // batch
                u = pair * 2 * batch + j

                if j % batch == 0:
                    wait_pos(pslot)
                wait_gather(bslot)

                @pl.loop(0, _CH)
                def _row(r, _bslot=bslot, _pslot=pslot):
                    @pl.loop(0, d_model, step=_L)
                    def _col(c):
                        v = pos[_pslot][r, pl.ds(c, _L)]
                        plsc.addupdate(rows[_bslot].at[r, pl.ds(c, _L)], v)

                fire_store(u, bslot)

                nslot = (bslot + 2) % _NBUF
                @pl.when(u >= 2)
                def _drain(_n=nslot):
                    wait_store(_n)

                @pl.when(u + 2 < n_units)
                def _prefetch(_u=u, _n=nslot):
                    fire_gather(_u + 2, _n)

                if bslot == batch - 1:
                    @pl.when(ck + 2 < n_chunks)
                    def _nextpos(_ck=ck, _ps=pslot):
                        fire_pos(_ck + 2, _ps)

        wait_store((n_units - 2) % _NBUF)
        wait_store((n_units - 1) % _NBUF)

    return k(table, idx, pos_enc)


def kernel(x, table, pos_enc):
    b, s = x.shape
    n = b * s
    d = table.shape[1]
    idx = x.reshape(n)
    out = _embed_sc(table, idx, pos_enc, b, s, d)
    return out.reshape(b, s, d)


# speedup vs baseline: 3.0146x; 2.0856x over previous
import functools

import jax
import jax.numpy as jnp
from jax import lax
from jax.experimental import pallas as pl
from jax.experimental.pallas import tpu as pltpu
from jax.experimental.pallas import tpu_sc as plsc

_NC = 2
_NS = 16
_NW = _NC * _NS
_CH = 16
_L = 16
_NBUF = 4


def _embed_sc(table, idx, pos_enc, batch, seq, d_model):
    mesh = plsc.VectorSubcoreMesh(core_axis_name="c", subcore_axis_name="s")
    n_rows = batch * seq
    per_w = seq // _NW
    n_chunks = per_w // _CH
    n_units = n_chunks * batch

    @functools.partial(
        pl.kernel,
        out_type=jax.ShapeDtypeStruct((n_rows, d_model), jnp.float32),
        mesh=mesh,
        scratch_types=(
            [pltpu.VMEM((batch, per_w), jnp.int32)]
            + [pltpu.VMEM((_CH, d_model), jnp.float32) for _ in range(_NBUF)]
            + [pltpu.VMEM((_CH, d_model), jnp.float32) for _ in range(2)]
            + [pltpu.SemaphoreType.DMA for _ in range(2 * _NBUF + 2)]
        ),
    )
    def k(table_hbm, idx_hbm, pos_hbm, out_hbm, idx_v,
          r0, r1, r2, r3, p0, p1,
          g0, g1, g2, g3, s0, s1, s2, s3, q0, q1):
        rows = (r0, r1, r2, r3)
        pos = (p0, p1)
        gsem = (g0, g1, g2, g3)
        ssem = (s0, s1, s2, s3)
        psem = (q0, q1)

        wid = lax.axis_index("s") * _NC + lax.axis_index("c")
        pbase = wid * per_w

        def fire_gather(u, bslot):
            ck = u // batch
            ib = bslot
            idx_ref = idx_v.at[ib, pl.ds(ck * _CH, _CH)]
            pltpu.make_async_copy(
                table_hbm.at[idx_ref], rows[bslot], gsem[bslot]).start()

        def wait_gather(bslot):
            pltpu.make_async_copy(
                table_hbm.at[idx_v.at[0, pl.ds(0, _CH)]],
                rows[bslot], gsem[bslot]).wait()

        def fire_pos(ck, pslot):
            pltpu.make_async_copy(
                pos_hbm.at[pl.ds(pbase + ck * _CH, _CH)],
                pos[pslot], psem[pslot]).start()

        def wait_pos(pslot):
            pltpu.make_async_copy(
                pos_hbm.at[pl.ds(pbase, _CH)], pos[pslot],
                psem[pslot]).wait()

        def fire_store(u, bslot):
            ck = u // batch
            b = bslot
            row0 = b * seq + pbase + ck * _CH
            pltpu.make_async_copy(
                rows[bslot], out_hbm.at[pl.ds(row0, _CH)],
                ssem[bslot]).start()

        def wait_store(bslot):
            pltpu.make_async_copy(
                rows[bslot], out_hbm.at[pl.ds(0, _CH)],
                ssem[bslot]).wait()

        for b in range(batch):
            pltpu.sync_copy(
                idx_hbm.at[pl.ds(b * seq + pbase, per_w)], idx_v.at[b])

        fire_pos(0, 0)
        fire_pos(1, 1)
        fire_gather(0, 0)
        fire_gather(1, 1)

        @pl.loop(0, n_chunks // 2)
        def _pair(pair):
            for j in range(2 * batch):
                ck = pair * 2 + (j // batch)
                bslot = j % batch
                pslot = j // batch
                u = pair * 2 * batch + j

                if j % batch == 0:
                    wait_pos(pslot)
                wait_gather(bslot)

                @pl.loop(0, _CH)
                def _row(r, _bslot=bslot, _pslot=pslot):
                    for c in range(0, d_model, _L):
                        v = pos[_pslot][r, pl.ds(c, _L)]
                        plsc.addupdate(rows[_bslot].at[r, pl.ds(c, _L)], v)

                fire_store(u, bslot)

                nslot = (bslot + 2) % _NBUF
                @pl.when(u >= 2)
                def _drain(_n=nslot):
                    wait_store(_n)

                @pl.when(u + 2 < n_units)
                def _prefetch(_u=u, _n=nslot):
                    fire_gather(_u + 2, _n)

                if bslot == batch - 1:
                    @pl.when(ck + 2 < n_chunks)
                    def _nextpos(_ck=ck, _ps=pslot):
                        fire_pos(_ck + 2, _ps)

        wait_store((n_units - 2) % _NBUF)
        wait_store((n_units - 1) % _NBUF)

    return k(table, idx, pos_enc)


def kernel(x, table, pos_enc):
    b, s = x.shape
    n = b * s
    d = table.shape[1]
    idx = x.reshape(n)
    out = _embed_sc(table, idx, pos_enc, b, s, d)
    return out.reshape(b, s, d)


# speedup vs baseline: 3.1933x; 1.0593x over previous
import functools

import jax
import jax.numpy as jnp
from jax import lax
from jax.experimental import pallas as pl
from jax.experimental.pallas import tpu as pltpu
from jax.experimental.pallas import tpu_sc as plsc

_NC = 2
_NS = 16
_NW = _NC * _NS
_CH = 8
_L = 16
_NBUF = 8
_LEAD = 4


def _embed_sc(table, idx, pos_enc, batch, seq, d_model):
    mesh = plsc.VectorSubcoreMesh(core_axis_name="c", subcore_axis_name="s")
    n_rows = batch * seq
    per_w = seq // _NW
    n_chunks = per_w // _CH
    n_units = n_chunks * batch

    @functools.partial(
        pl.kernel,
        out_type=jax.ShapeDtypeStruct((n_rows, d_model), jnp.float32),
        mesh=mesh,
        scratch_types=(
            [pltpu.VMEM((batch, per_w), jnp.int32)]
            + [pltpu.VMEM((_CH, d_model), jnp.float32) for _ in range(_NBUF)]
            + [pltpu.VMEM((_CH, d_model), jnp.float32) for _ in range(2)]
            + [pltpu.SemaphoreType.DMA for _ in range(2 * _NBUF + 2)]
        ),
    )
    def k(table_hbm, idx_hbm, pos_hbm, out_hbm, idx_v, *scr):
        rows = scr[:_NBUF]
        pos = scr[_NBUF:_NBUF + 2]
        gsem = scr[_NBUF + 2:2 * _NBUF + 2]
        ssem = scr[2 * _NBUF + 2:3 * _NBUF + 2]
        psem = scr[3 * _NBUF + 2:]

        wid = lax.axis_index("s") * _NC + lax.axis_index("c")
        pbase = wid * per_w

        def fire_gather(u, slot):
            ck = u // batch
            ib = u % batch if isinstance(u, int) else slot % batch
            idx_ref = idx_v.at[ib, pl.ds(ck * _CH, _CH)]
            pltpu.make_async_copy(
                table_hbm.at[idx_ref], rows[slot], gsem[slot]).start()

        def wait_gather(slot):
            pltpu.make_async_copy(
                table_hbm.at[idx_v.at[0, pl.ds(0, _CH)]],
                rows[slot], gsem[slot]).wait()

        def fire_pos(ck, pslot):
            pltpu.make_async_copy(
                pos_hbm.at[pl.ds(pbase + ck * _CH, _CH)],
                pos[pslot], psem[pslot]).start()

        def wait_pos(pslot):
            pltpu.make_async_copy(
                pos_hbm.at[pl.ds(pbase, _CH)], pos[pslot],
                psem[pslot]).wait()

        def fire_store(u, slot, b):
            ck = u // batch
            row0 = b * seq + pbase + ck * _CH
            pltpu.make_async_copy(
                rows[slot], out_hbm.at[pl.ds(row0, _CH)],
                ssem[slot]).start()

        def wait_store(slot):
            pltpu.make_async_copy(
                rows[slot], out_hbm.at[pl.ds(0, _CH)],
                ssem[slot]).wait()

        for b in range(batch):
            pltpu.sync_copy(
                idx_hbm.at[pl.ds(b * seq + pbase, per_w)], idx_v.at[b])

        fire_pos(0, 0)
        fire_pos(1, 1)
        for u0 in range(_LEAD):
            fire_gather(u0, u0 % _NBUF)

        @pl.loop(0, n_chunks // 2)
        def _pair(pair):
            for j in range(2 * batch):
                ck = pair * 2 + (j // batch)
                slot = j
                bi = j % batch
                pslot = j // batch
                u = pair * 2 * batch + j

                if bi == 0:
                    wait_pos(pslot)
                wait_gather(slot)

                @pl.loop(0, _CH)
                def _row(r, _slot=slot, _ps=pslot):
                    for c in range(0, d_model, _L):
                        v = pos[_ps][r, pl.ds(c, _L)]
                        plsc.addupdate(rows[_slot].at[r, pl.ds(c, _L)], v)

                fire_store(u, slot, bi)

                nslot = (slot + _LEAD) % _NBUF
                @pl.when(u >= _LEAD)
                def _drain(_n=nslot):
                    wait_store(_n)

                @pl.when(u + _LEAD < n_units)
                def _prefetch(_u=u, _n=nslot):
                    fire_gather(_u + _LEAD, _n)

                if bi == batch - 1:
                    @pl.when(ck + 2 < n_chunks)
                    def _nextpos(_ck=ck, _ps=pslot):
                        fire_pos(_ck + 2, _ps)

        for u0 in range(n_units - _LEAD, n_units):
            wait_store(u0 % _NBUF)

    return k(table, idx, pos_enc)


def kernel(x, table, pos_enc):
    b, s = x.shape
    n = b * s
    d = table.shape[1]
    idx = x.reshape(n)
    out = _embed_sc(table, idx, pos_enc, b, s, d)
    return out.reshape(b, s, d)
